# baseline (speedup 1.0000x reference)
import functools
import math

import jax
import jax.numpy as jnp
from jax.experimental import pallas as pl
from jax.experimental.pallas import tpu as pltpu

E = 8
D = 1024
F = 4096
K = 2
T = 2048
C = 320
ALC = 0.01

_HI = jax.lax.Precision.HIGHEST


def _dot(a, b, dims, precision=_HI):
    return jax.lax.dot_general(
        a, b, (dims, ((), ())), precision=precision,
        preferred_element_type=jnp.float32)


def _router_kernel(x_ref, wg_ref, disp_ref, yidx_ref, w_ref, aux_ref):
    x = x_ref[...]
    wg = wg_ref[...]
    logits = jax.lax.dot_general(
        x.astype(jnp.bfloat16), wg.astype(jnp.bfloat16),
        (((1,), (1,)), ((), ())),
        preferred_element_type=jnp.float32)

    lane = jax.lax.broadcasted_iota(jnp.int32, (T, E), 1)
    m1 = jnp.max(logits, axis=1, keepdims=True)
    i1 = jnp.min(jnp.where(logits == m1, lane, E), axis=1)
    oh1 = (lane == i1[:, None]).astype(jnp.float32)
    masked = jnp.where(lane == i1[:, None], -jnp.inf, logits)
    m2 = jnp.max(masked, axis=1, keepdims=True)
    i2 = jnp.min(jnp.where(masked == m2, lane, E), axis=1)
    oh2 = (lane == i2[:, None]).astype(jnp.float32)

    p = jnp.exp(logits - m1)
    probs = p / jnp.sum(p, axis=1, keepdims=True)
    meanprobs = jnp.mean(probs, axis=0)
    counts = jnp.sum(oh1 + oh2, axis=0)
    aux = ALC * E * jnp.sum((counts / T) * meanprobs)
    aux_ref[...] = aux[None, None]

    oh12 = oh1 + oh2
    CH = 512
    tri = (jax.lax.broadcasted_iota(jnp.int32, (CH, CH), 1)
           <= jax.lax.broadcasted_iota(jnp.int32, (CH, CH), 0)
           ).astype(jnp.float32)
    carry = jnp.zeros((1, E), jnp.float32)
    parts = []
    for c in range(T // CH):
        blk = oh12[c * CH:(c + 1) * CH]
        sc = _dot(tri, blk, ((1,), (0,))) + carry
        carry = carry + jnp.sum(blk, axis=0, keepdims=True)
        parts.append(sc)
    s_inc = jnp.concatenate(parts, axis=0)

    pos1 = jnp.sum(oh1 * s_inc, axis=1) - 1.0
    pos2 = jnp.sum(oh2 * s_inc, axis=1) - 1.0

    e2 = jnp.exp(m2[:, 0] - m1[:, 0])
    w1 = 1.0 / (1.0 + e2)
    w2 = e2 / (1.0 + e2)
    k1 = (pos1 < C).astype(jnp.float32)
    k2 = (pos2 < C).astype(jnp.float32)
    wk1 = w1 * k1
    wk2 = w2 * k2
    s = wk1 + wk2 + 1e-8
    wn1 = wk1 / s
    wn2 = wk2 / s
    w_ref[0, :] = wn1
    w_ref[1, :] = wn2

    i1f = i1.astype(jnp.float32)
    i2f = i2.astype(jnp.float32)
    y1 = (i1f * C + pos1) * k1
    y2 = (i2f * C + pos2) * k2
    yidx_ref[0, :] = y1.astype(jnp.int32)
    yidx_ref[1, :] = y2.astype(jnp.int32)

    tvec = jax.lax.broadcasted_iota(jnp.int32, (1, T), 1).astype(jnp.float32)
    p_iota = jax.lax.broadcasted_iota(jnp.int32, (C, T), 0).astype(jnp.float32)
    for e in range(E):
        m1e = (i1 == e).astype(jnp.float32)
        m2e = (i2 == e).astype(jnp.float32)
        a1 = (p_iota == pos1[None, :]).astype(jnp.float32) * m1e[None, :]
        a2 = (p_iota == pos2[None, :]).astype(jnp.float32) * m2e[None, :]
        disp_ref[e, :] = jnp.sum((a1 + a2) * tvec, axis=1).astype(jnp.int32)


def _ffn_kernel(x_ref, disp_ref, w1_ref, w3_ref, w2_ref, y_ref, xe_ref):
    f = pl.program_id(1)

    @pl.when(f == 0)
    def _gather():
        d = disp_ref[0, 0, :]
        tok = jax.lax.broadcasted_iota(jnp.int32, (C, T), 1)
        g = (d[:, None] == tok).astype(jnp.float32)
        xe_ref[...] = _dot(g, x_ref[...], ((1,), (0,)))

    xe = xe_ref[...]
    h1 = _dot(xe, w1_ref[0], ((1,), (1,)))
    h3 = _dot(xe, w3_ref[0], ((1,), (1,)))
    h = h1 * jax.nn.sigmoid(h1) * h3
    acc = _dot(h, w2_ref[0], ((1,), (1,)))

    @pl.when(f == 0)
    def _init():
        y_ref[0] = acc

    @pl.when(f > 0)
    def _acc():
        y_ref[0] = y_ref[0] + acc


def _combine_kernel(yf_ref, yi1_ref, yi2_ref, w1_ref, w2_ref, out_ref):
    tt = out_ref.shape[0]
    y1 = yi1_ref[0, 0, :]
    y2 = yi2_ref[0, 0, :]
    w1v = w1_ref[0, 0, :]
    w2v = w2_ref[0, 0, :]
    slots = jax.lax.broadcasted_iota(jnp.int32, (tt, E * C), 1)
    coeff = ((y1[:, None] == slots).astype(jnp.float32) * w1v[:, None]
             + (y2[:, None] == slots).astype(jnp.float32) * w2v[:, None])
    out_ref[...] = _dot(coeff, yf_ref[...], ((1,), (0,)))


@jax.jit
def kernel(x, W_gate, W1, W3, W2):
    bsz, seq_len, dim = x.shape
    x_flat = x.reshape(T, D)

    disp, yidx, wn, aux = pl.pallas_call(
        _router_kernel,
        out_shape=[
            jax.ShapeDtypeStruct((E, C), jnp.int32),
            jax.ShapeDtypeStruct((K, T), jnp.int32),
            jax.ShapeDtypeStruct((K, T), jnp.float32),
            jax.ShapeDtypeStruct((1, 1), jnp.float32),
        ],
    )(x_flat, W_gate)

    FT = 1024
    NF = F // FT
    disp3 = disp.reshape(E, 1, C)
    y = pl.pallas_call(
        _ffn_kernel,
        grid=(E, NF),
        in_specs=[
            pl.BlockSpec((T, D), lambda e, f: (0, 0)),
            pl.BlockSpec((1, 1, C), lambda e, f: (e, 0, 0)),
            pl.BlockSpec((1, FT, D), lambda e, f: (e, f, 0)),
            pl.BlockSpec((1, FT, D), lambda e, f: (e, f, 0)),
            pl.BlockSpec((1, D, FT), lambda e, f: (e, 0, f)),
        ],
        out_specs=pl.BlockSpec((1, C, D), lambda e, f: (e, 0, 0)),
        out_shape=jax.ShapeDtypeStruct((E, C, D), jnp.float32),
        scratch_shapes=[pltpu.VMEM((C, D), jnp.float32)],
        compiler_params=pltpu.CompilerParams(
            dimension_semantics=("arbitrary", "arbitrary")),
    )(x_flat, disp3, W1, W3, W2)

    TT = 256
    NT = T // TT
    yf = y.reshape(E * C, D)
    yi1 = yidx[0].reshape(NT, 1, TT)
    yi2 = yidx[1].reshape(NT, 1, TT)
    wn1 = wn[0].reshape(NT, 1, TT)
    wn2 = wn[1].reshape(NT, 1, TT)
    out = pl.pallas_call(
        _combine_kernel,
        grid=(NT,),
        in_specs=[
            pl.BlockSpec((E * C, D), lambda t: (0, 0)),
            pl.BlockSpec((1, 1, TT), lambda t: (t, 0, 0)),
            pl.BlockSpec((1, 1, TT), lambda t: (t, 0, 0)),
            pl.BlockSpec((1, 1, TT), lambda t: (t, 0, 0)),
            pl.BlockSpec((1, 1, TT), lambda t: (t, 0, 0)),
        ],
        out_specs=pl.BlockSpec((TT, D), lambda t: (t, 0)),
        out_shape=jax.ShapeDtypeStruct((T, D), jnp.float32),
        compiler_params=pltpu.CompilerParams(
            dimension_semantics=("arbitrary",)),
    )(yf, yi1, yi2, wn1, wn2)

    return out.reshape(bsz, seq_len, dim), aux[0, 0]


# speedup vs baseline: 3.2028x; 3.2028x over previous
import functools
import math

import jax
import jax.numpy as jnp
from jax.experimental import pallas as pl
from jax.experimental.pallas import tpu as pltpu

E = 8
D = 1024
F = 4096
K = 2
T = 2048
C = 320
ALC = 0.01

_HI = jax.lax.Precision.HIGHEST


def _dot(a, b, dims, precision=_HI):
    return jax.lax.dot_general(
        a, b, (dims, ((), ())), precision=precision,
        preferred_element_type=jnp.float32)


def _bdot(a, b, dims):
    return jax.lax.dot_general(
        a.astype(jnp.bfloat16), b.astype(jnp.bfloat16), (dims, ((), ())),
        preferred_element_type=jnp.float32)


def _router_kernel(x_ref, wg_ref, disp_ref, yidx_ref, w_ref, aux_ref):
    x = x_ref[...]
    wg = wg_ref[...]
    logits = _bdot(x, wg, ((1,), (1,)))

    lane = jax.lax.broadcasted_iota(jnp.int32, (T, E), 1)
    m1 = jnp.max(logits, axis=1, keepdims=True)
    i1 = jnp.min(jnp.where(logits == m1, lane, E), axis=1)
    oh1 = (lane == i1[:, None]).astype(jnp.float32)
    masked = jnp.where(lane == i1[:, None], -jnp.inf, logits)
    m2 = jnp.max(masked, axis=1, keepdims=True)
    i2 = jnp.min(jnp.where(masked == m2, lane, E), axis=1)
    oh2 = (lane == i2[:, None]).astype(jnp.float32)

    p = jnp.exp(logits - m1)
    probs = p / jnp.sum(p, axis=1, keepdims=True)
    meanprobs = jnp.mean(probs, axis=0)
    counts = jnp.sum(oh1 + oh2, axis=0)
    aux = ALC * E * jnp.sum((counts / T) * meanprobs)
    aux_ref[...] = aux[None, None]

    oh12 = oh1 + oh2
    CH = 512
    tri = (jax.lax.broadcasted_iota(jnp.int32, (CH, CH), 1)
           <= jax.lax.broadcasted_iota(jnp.int32, (CH, CH), 0)
           ).astype(jnp.float32)
    carry = jnp.zeros((1, E), jnp.float32)
    parts = []
    for c in range(T // CH):
        blk = oh12[c * CH:(c + 1) * CH]
        sc = _dot(tri, blk, ((1,), (0,))) + carry
        carry = carry + jnp.sum(blk, axis=0, keepdims=True)
        parts.append(sc)
    s_inc = jnp.concatenate(parts, axis=0)

    pos1 = jnp.sum(oh1 * s_inc, axis=1) - 1.0
    pos2 = jnp.sum(oh2 * s_inc, axis=1) - 1.0

    e2 = jnp.exp(m2[:, 0] - m1[:, 0])
    w1 = 1.0 / (1.0 + e2)
    w2 = e2 / (1.0 + e2)
    k1 = (pos1 < C).astype(jnp.float32)
    k2 = (pos2 < C).astype(jnp.float32)
    wk1 = w1 * k1
    wk2 = w2 * k2
    s = wk1 + wk2 + 1e-8
    wn1 = wk1 / s
    wn2 = wk2 / s
    w_ref[0, :] = wn1
    w_ref[1, :] = wn2

    i1f = i1.astype(jnp.float32)
    i2f = i2.astype(jnp.float32)
    y1 = (i1f * C + pos1) * k1
    y2 = (i2f * C + pos2) * k2
    yidx_ref[0, :] = y1.astype(jnp.int32)
    yidx_ref[1, :] = y2.astype(jnp.int32)

    tvec = jax.lax.broadcasted_iota(jnp.int32, (1, T), 1).astype(jnp.float32)
    p_iota = jax.lax.broadcasted_iota(jnp.int32, (C, T), 0).astype(jnp.float32)
    for e in range(E):
        m1e = (i1 == e).astype(jnp.float32)
        m2e = (i2 == e).astype(jnp.float32)
        a1 = (p_iota == pos1[None, :]).astype(jnp.float32) * m1e[None, :]
        a2 = (p_iota == pos2[None, :]).astype(jnp.float32) * m2e[None, :]
        disp_ref[e, :] = jnp.sum((a1 + a2) * tvec, axis=1).astype(jnp.int32)


def _ffn_kernel(x_ref, disp_ref, w1_ref, w3_ref, w2_ref, y_ref, xe_ref):
    f = pl.program_id(1)

    @pl.when(f == 0)
    def _gather():
        d = disp_ref[0, 0, :]
        tok = jax.lax.broadcasted_iota(jnp.int32, (C, T), 1)
        g = (d[:, None] == tok).astype(jnp.bfloat16)
        xe_ref[...] = jax.lax.dot_general(
            g, x_ref[...].astype(jnp.bfloat16), ((((1,), (0,))), ((), ())),
            preferred_element_type=jnp.float32)

    xe = xe_ref[...]
    h1 = _bdot(xe, w1_ref[0], ((1,), (1,)))
    h3 = _bdot(xe, w3_ref[0], ((1,), (1,)))
    h = h1 * jax.nn.sigmoid(h1) * h3
    acc = _bdot(h, w2_ref[0], ((1,), (1,)))

    @pl.when(f == 0)
    def _init():
        y_ref[0] = acc

    @pl.when(f > 0)
    def _acc():
        y_ref[0] = y_ref[0] + acc


def _combine_kernel(yf_ref, yi1_ref, yi2_ref, w1_ref, w2_ref, out_ref):
    tt = out_ref.shape[0]
    y1 = yi1_ref[0, 0, :]
    y2 = yi2_ref[0, 0, :]
    w1v = w1_ref[0, 0, :]
    w2v = w2_ref[0, 0, :]
    slots = jax.lax.broadcasted_iota(jnp.int32, (tt, E * C), 1)
    coeff = ((y1[:, None] == slots).astype(jnp.float32) * w1v[:, None]
             + (y2[:, None] == slots).astype(jnp.float32) * w2v[:, None])
    out_ref[...] = _bdot(coeff, yf_ref[...], ((1,), (0,)))


@jax.jit
def kernel(x, W_gate, W1, W3, W2):
    bsz, seq_len, dim = x.shape
    x_flat = x.reshape(T, D)

    disp, yidx, wn, aux = pl.pallas_call(
        _router_kernel,
        out_shape=[
            jax.ShapeDtypeStruct((E, C), jnp.int32),
            jax.ShapeDtypeStruct((K, T), jnp.int32),
            jax.ShapeDtypeStruct((K, T), jnp.float32),
            jax.ShapeDtypeStruct((1, 1), jnp.float32),
        ],
    )(x_flat, W_gate)

    FT = 1024
    NF = F // FT
    disp3 = disp.reshape(E, 1, C)
    y = pl.pallas_call(
        _ffn_kernel,
        grid=(E, NF),
        in_specs=[
            pl.BlockSpec((T, D), lambda e, f: (0, 0)),
            pl.BlockSpec((1, 1, C), lambda e, f: (e, 0, 0)),
            pl.BlockSpec((1, FT, D), lambda e, f: (e, f, 0)),
            pl.BlockSpec((1, FT, D), lambda e, f: (e, f, 0)),
            pl.BlockSpec((1, D, FT), lambda e, f: (e, 0, f)),
        ],
        out_specs=pl.BlockSpec((1, C, D), lambda e, f: (e, 0, 0)),
        out_shape=jax.ShapeDtypeStruct((E, C, D), jnp.float32),
        scratch_shapes=[pltpu.VMEM((C, D), jnp.float32)],
        compiler_params=pltpu.CompilerParams(
            dimension_semantics=("arbitrary", "arbitrary")),
    )(x_flat, disp3, W1, W3, W2)

    TT = 256
    NT = T // TT
    yf = y.reshape(E * C, D)
    yi1 = yidx[0].reshape(NT, 1, TT)
    yi2 = yidx[1].reshape(NT, 1, TT)
    wn1 = wn[0].reshape(NT, 1, TT)
    wn2 = wn[1].reshape(NT, 1, TT)
    out = pl.pallas_call(
        _combine_kernel,
        grid=(NT,),
        in_specs=[
            pl.BlockSpec((E * C, D), lambda t: (0, 0)),
            pl.BlockSpec((1, 1, TT), lambda t: (t, 0, 0)),
            pl.BlockSpec((1, 1, TT), lambda t: (t, 0, 0)),
            pl.BlockSpec((1, 1, TT), lambda t: (t, 0, 0)),
            pl.BlockSpec((1, 1, TT), lambda t: (t, 0, 0)),
        ],
        out_specs=pl.BlockSpec((TT, D), lambda t: (t, 0)),
        out_shape=jax.ShapeDtypeStruct((T, D), jnp.float32),
        compiler_params=pltpu.CompilerParams(
            dimension_semantics=("arbitrary",)),
    )(yf, yi1, yi2, wn1, wn2)

    return out.reshape(bsz, seq_len, dim), aux[0, 0]


# speedup vs baseline: 3.4136x; 1.0658x over previous
import functools
import math

import jax
import jax.numpy as jnp
from jax.experimental import pallas as pl
from jax.experimental.pallas import tpu as pltpu

E = 8
D = 1024
F = 4096
K = 2
T = 2048
C = 320
ALC = 0.01

FT = 1024
NF = F // FT


def _bdot(a, b, dims):
    return jax.lax.dot_general(
        a.astype(jnp.bfloat16), b.astype(jnp.bfloat16), (dims, ((), ())),
        preferred_element_type=jnp.float32)


def _route(x, wg):
    logits = _bdot(x, wg, ((1,), (1,)))

    lane = jax.lax.broadcasted_iota(jnp.int32, (T, E), 1)
    m1 = jnp.max(logits, axis=1, keepdims=True)
    i1 = jnp.min(jnp.where(logits == m1, lane, E), axis=1)
    oh1 = (lane == i1[:, None]).astype(jnp.float32)
    masked = jnp.where(lane == i1[:, None], -jnp.inf, logits)
    m2 = jnp.max(masked, axis=1, keepdims=True)
    i2 = jnp.min(jnp.where(masked == m2, lane, E), axis=1)
    oh2 = (lane == i2[:, None]).astype(jnp.float32)

    p = jnp.exp(logits - m1)
    probs = p / jnp.sum(p, axis=1, keepdims=True)
    meanprobs = jnp.mean(probs, axis=0)
    counts = jnp.sum(oh1 + oh2, axis=0)
    aux = ALC * E * jnp.sum((counts / T) * meanprobs)

    oh12 = oh1 + oh2
    CH = 512
    tri = (jax.lax.broadcasted_iota(jnp.int32, (CH, CH), 1)
           <= jax.lax.broadcasted_iota(jnp.int32, (CH, CH), 0)
           ).astype(jnp.float32)
    carry = jnp.zeros((1, E), jnp.float32)
    parts = []
    for c in range(T // CH):
        blk = oh12[c * CH:(c + 1) * CH]
        sc = jax.lax.dot_general(
            tri, blk, ((((1,), (0,))), ((), ())),
            precision=jax.lax.Precision.HIGHEST,
            preferred_element_type=jnp.float32) + carry
        carry = carry + jnp.sum(blk, axis=0, keepdims=True)
        parts.append(sc)
    s_inc = jnp.concatenate(parts, axis=0)

    pos1 = jnp.sum(oh1 * s_inc, axis=1) - 1.0
    pos2 = jnp.sum(oh2 * s_inc, axis=1) - 1.0

    e2 = jnp.exp(m2[:, 0] - m1[:, 0])
    w1 = 1.0 / (1.0 + e2)
    w2 = e2 / (1.0 + e2)
    k1 = pos1 < C
    k2 = pos2 < C
    wk1 = jnp.where(k1, w1, 0.0)
    wk2 = jnp.where(k2, w2, 0.0)
    s = wk1 + wk2 + 1e-8
    wn1 = wk1 / s
    wn2 = wk2 / s

    y1 = jnp.where(k1, i1.astype(jnp.float32) * C + pos1, float(E * C))
    y2 = jnp.where(k2, i2.astype(jnp.float32) * C + pos2, float(E * C))
    return y1, y2, wn1, wn2, aux


def _moe_kernel(x_ref, wg_ref, w1_ref, w3_ref, w2_ref,
                out_ref, aux_ref, rt_ref, rtt_ref, xe_ref, yc_ref):
    e = pl.program_id(0)
    f = pl.program_id(1)

    @pl.when((e == 0) & (f == 0))
    def _router():
        y1, y2, wn1, wn2, aux = _route(x_ref[...], wg_ref[...])
        aux_ref[...] = aux[None, None]
        rt_ref[0, :] = y1
        rt_ref[1, :] = y2
        rtt_ref[:, 0:1] = y1[:, None]
        rtt_ref[:, 1:2] = y2[:, None]
        rtt_ref[:, 2:3] = wn1[:, None]
        rtt_ref[:, 3:4] = wn2[:, None]

    @pl.when(f == 0)
    def _gather():
        y1r = rt_ref[0:1, :].astype(jnp.int32)
        y2r = rt_ref[1:2, :].astype(jnp.int32)
        slots = e * C + jax.lax.broadcasted_iota(jnp.int32, (C, T), 0)
        g = ((slots == y1r) | (slots == y2r)).astype(jnp.bfloat16)
        xe_ref[...] = jax.lax.dot_general(
            g, x_ref[...].astype(jnp.bfloat16), ((((1,), (0,))), ((), ())),
            preferred_element_type=jnp.float32)

    xe = xe_ref[...]
    h1 = _bdot(xe, w1_ref[0], ((1,), (1,)))
    h3 = _bdot(xe, w3_ref[0], ((1,), (1,)))
    h = h1 * jax.nn.sigmoid(h1) * h3
    acc = _bdot(h, w2_ref[0], ((1,), (1,)))

    @pl.when(f == 0)
    def _init_yc():
        yc_ref[...] = acc

    @pl.when(f > 0)
    def _acc_yc():
        yc_ref[...] = yc_ref[...] + acc

    @pl.when(f == NF - 1)
    def _scatter():
        y1c = rtt_ref[:, 0:1].astype(jnp.int32)
        y2c = rtt_ref[:, 1:2].astype(jnp.int32)
        w1c = rtt_ref[:, 2:3]
        w2c = rtt_ref[:, 3:4]
        slots = e * C + jax.lax.broadcasted_iota(jnp.int32, (T, C), 1)
        coeff = ((slots == y1c).astype(jnp.float32) * w1c
                 + (slots == y2c).astype(jnp.float32) * w2c)
        contrib = _bdot(coeff, yc_ref[...], ((1,), (0,)))

        @pl.when(e == 0)
        def _():
            out_ref[...] = contrib

        @pl.when(e > 0)
        def _():
            out_ref[...] = out_ref[...] + contrib


@jax.jit
def kernel(x, W_gate, W1, W3, W2):
    bsz, seq_len, dim = x.shape
    x_flat = x.reshape(T, D)

    out, aux = pl.pallas_call(
        _moe_kernel,
        grid=(E, NF),
        in_specs=[
            pl.BlockSpec((T, D), lambda e, f: (0, 0)),
            pl.BlockSpec((E, D), lambda e, f: (0, 0)),
            pl.BlockSpec((1, FT, D), lambda e, f: (e, f, 0)),
            pl.BlockSpec((1, FT, D), lambda e, f: (e, f, 0)),
            pl.BlockSpec((1, D, FT), lambda e, f: (e, 0, f)),
        ],
        out_specs=[
            pl.BlockSpec((T, D), lambda e, f: (0, 0)),
            pl.BlockSpec((1, 1), lambda e, f: (0, 0)),
        ],
        out_shape=[
            jax.ShapeDtypeStruct((T, D), jnp.float32),
            jax.ShapeDtypeStruct((1, 1), jnp.float32),
        ],
        scratch_shapes=[
            pltpu.VMEM((K, T), jnp.float32),
            pltpu.VMEM((T, 8), jnp.float32),
            pltpu.VMEM((C, D), jnp.float32),
            pltpu.VMEM((C, D), jnp.float32),
        ],
        compiler_params=pltpu.CompilerParams(
            dimension_semantics=("arbitrary", "arbitrary")),
    )(x_flat, W_gate, W1, W3, W2)

    return out.reshape(bsz, seq_len, dim), aux[0, 0]
